# baseline (speedup 1.0000x reference)
import functools

import jax
import jax.numpy as jnp
from jax import lax
from jax.experimental import pallas as pl
from jax.experimental.pallas import tpu as pltpu
from jax.experimental.pallas import tpu_sc as plsc

DIM = 128
SEQ = 2048
BATCH = 32
N = BATCH * SEQ
EPS = 1e-5

NUM_CORES = 2
NUM_SUBCORES = 16
NW = NUM_CORES * NUM_SUBCORES
CHUNK = 128

P = 4
B_SLICE = BATCH // P
NS = B_SLICE * SEQ
PER_W = NS // NW
CPW = PER_W // CHUNK

R = 1024
SB = SEQ // R
NSB = NS // R


_sc_mesh = plsc.VectorSubcoreMesh(core_axis_name="c", subcore_axis_name="s")


def _make_tok_gather(k):
    @functools.partial(
        pl.kernel,
        mesh=_sc_mesh,
        out_type=jax.ShapeDtypeStruct((NS, DIM), jnp.float32),
        scratch_types=[
            pltpu.VMEM((CPW, CHUNK), jnp.int32),
            pltpu.VMEM((PER_W, DIM), jnp.float32),
        ]
        + [pltpu.SemaphoreType.DMA] * (2 * CPW),
    )
    def _tok_gather(idx_hbm, table_hbm, out_hbm, idx_v, buf, *sems):
        gsems, wsems = sems[:CPW], sems[CPW:]
        wid = lax.axis_index("s") * NUM_CORES + lax.axis_index("c")
        pltpu.sync_copy(idx_hbm.at[k].at[wid], idx_v)
        base = wid * PER_W

        for j in range(CPW):
            pltpu.async_copy(
                table_hbm.at[idx_v.at[j]],
                buf.at[pl.ds(j * CHUNK, CHUNK)],
                gsems[j],
            )
        for j in range(CPW):
            pltpu.make_async_copy(
                table_hbm.at[idx_v.at[j]],
                buf.at[pl.ds(j * CHUNK, CHUNK)],
                gsems[j],
            ).wait()
            pltpu.async_copy(
                buf.at[pl.ds(j * CHUNK, CHUNK)],
                out_hbm.at[pl.ds(base + j * CHUNK, CHUNK)],
                wsems[j],
            )
        for j in range(CPW):
            pltpu.make_async_copy(
                buf.at[pl.ds(j * CHUNK, CHUNK)],
                out_hbm.at[pl.ds(base + j * CHUNK, CHUNK)],
                wsems[j],
            ).wait()

    return _tok_gather


_tok_gathers = [_make_tok_gather(k) for k in range(P)]


def _ln_body(tok_ref, pos_ref, seg_ref, segtab_ref, m_ref, gamma_ref, beta_ref,
             out_ref):
    t = tok_ref[...]
    p = pos_ref[...]
    sf = seg_ref[0, 0].reshape(R, 1)
    s0 = segtab_ref[0:1, :]
    d = segtab_ref[1:2, :] - s0
    e = t + p + s0 + sf * d
    m = m_ref[...]
    mean_b = lax.dot(e, m, precision=lax.Precision.DEFAULT)
    e2_b = lax.dot(e * e, m, precision=lax.Precision.DEFAULT)
    var = e2_b - mean_b * mean_b
    out_ref[...] = (
        (e - mean_b) * lax.rsqrt(var + EPS) * gamma_ref[...] + beta_ref[...]
    )


def _ln_body_alias(tok_ref, pos_ref, seg_ref, segtab_ref, m_ref, gamma_ref,
                   beta_ref, prev_ref, out_ref):
    del prev_ref
    _ln_body(tok_ref, pos_ref, seg_ref, segtab_ref, m_ref, gamma_ref, beta_ref,
             out_ref)


def _ln_in_specs(k):
    return [
        pl.BlockSpec((R, DIM), lambda sb, b: (b * SB + sb, 0)),
        pl.BlockSpec((R, DIM), lambda sb, b: (sb, 0)),
        pl.BlockSpec((1, 1, R), lambda sb, b, k=k: (k * NSB + b * SB + sb, 0, 0)),
        pl.BlockSpec((2, DIM), lambda sb, b: (0, 0)),
        pl.BlockSpec((DIM, DIM), lambda sb, b: (0, 0)),
        pl.BlockSpec((1, DIM), lambda sb, b: (0, 0)),
        pl.BlockSpec((1, DIM), lambda sb, b: (0, 0)),
    ]


def _ln_slice(k, tok_e_s, pos_table, seg_f, seg_table, mmat, gamma2, beta2, prev):
    out_spec = pl.BlockSpec(
        (R, DIM), lambda sb, b, k=k: (k * NSB + b * SB + sb, 0)
    )
    out_shape = jax.ShapeDtypeStruct((N, DIM), jnp.float32)
    args = (tok_e_s, pos_table, seg_f, seg_table, mmat, gamma2, beta2)
    if prev is None:
        return pl.pallas_call(
            _ln_body,
            grid=(SB, B_SLICE),
            in_specs=_ln_in_specs(k),
            out_specs=out_spec,
            out_shape=out_shape,
        )(*args)
    return pl.pallas_call(
        _ln_body_alias,
        grid=(SB, B_SLICE),
        in_specs=_ln_in_specs(k) + [pl.BlockSpec(memory_space=pl.ANY)],
        out_specs=out_spec,
        out_shape=out_shape,
        input_output_aliases={7: 0},
    )(*args, prev)


def kernel(x, seg, tok_table, pos_table, seg_table, gamma, beta):
    idx = x.astype(jnp.int32).reshape(P, NW, CPW, CHUNK)
    seg_f = seg.astype(jnp.float32).reshape(P * NSB, 1, R)
    gamma2 = gamma.reshape(1, DIM)
    beta2 = beta.reshape(1, DIM)
    mmat = jnp.full((DIM, DIM), 1.0 / DIM, dtype=jnp.float32)

    tok_slices = [_tok_gathers[k](idx, tok_table) for k in range(P)]
    out = None
    for k in range(P):
        out = _ln_slice(
            k, tok_slices[k], pos_table, seg_f, seg_table, mmat, gamma2,
            beta2, out
        )
    return out.reshape(BATCH, SEQ, DIM)


# speedup vs baseline: 1.1905x; 1.1905x over previous
import functools

import jax
import jax.numpy as jnp
from jax import lax
from jax.experimental import pallas as pl
from jax.experimental.pallas import tpu as pltpu
from jax.experimental.pallas import tpu_sc as plsc

DIM = 128
SEQ = 2048
BATCH = 32
N = BATCH * SEQ
EPS = 1e-5

NUM_CORES = 2
NUM_SUBCORES = 16
NW = NUM_CORES * NUM_SUBCORES
CHUNK = 128

P = 4
B_SLICE = BATCH // P
NS = B_SLICE * SEQ
PER_W = NS // NW
CPW = PER_W // CHUNK

R = 2048
SB = SEQ // R
NSB = NS // R


_sc_mesh = plsc.VectorSubcoreMesh(core_axis_name="c", subcore_axis_name="s")


def _make_tok_gather(k):
    @functools.partial(
        pl.kernel,
        mesh=_sc_mesh,
        out_type=jax.ShapeDtypeStruct((NS, DIM), jnp.float32),
        scratch_types=[
            pltpu.VMEM((CPW, CHUNK), jnp.int32),
            pltpu.VMEM((PER_W, DIM), jnp.float32),
        ]
        + [pltpu.SemaphoreType.DMA] * (2 * CPW),
    )
    def _tok_gather(idx_hbm, table_hbm, out_hbm, idx_v, buf, *sems):
        gsems, wsems = sems[:CPW], sems[CPW:]
        wid = lax.axis_index("s") * NUM_CORES + lax.axis_index("c")
        pltpu.sync_copy(idx_hbm.at[k].at[wid], idx_v)
        base = wid * PER_W

        for j in range(CPW):
            pltpu.async_copy(
                table_hbm.at[idx_v.at[j]],
                buf.at[pl.ds(j * CHUNK, CHUNK)],
                gsems[j],
            )
        for j in range(CPW):
            pltpu.make_async_copy(
                table_hbm.at[idx_v.at[j]],
                buf.at[pl.ds(j * CHUNK, CHUNK)],
                gsems[j],
            ).wait()
            pltpu.async_copy(
                buf.at[pl.ds(j * CHUNK, CHUNK)],
                out_hbm.at[pl.ds(base + j * CHUNK, CHUNK)],
                wsems[j],
            )
        for j in range(CPW):
            pltpu.make_async_copy(
                buf.at[pl.ds(j * CHUNK, CHUNK)],
                out_hbm.at[pl.ds(base + j * CHUNK, CHUNK)],
                wsems[j],
            ).wait()

    return _tok_gather


_tok_gathers = [_make_tok_gather(k) for k in range(P)]


def _ln_body(tok_ref, pos_ref, seg_ref, segtab_ref, m_ref, gamma_ref, beta_ref,
             out_ref):
    t = tok_ref[...]
    p = pos_ref[...]
    sf = seg_ref[0, 0].reshape(R, 1)
    s0 = segtab_ref[0:1, :]
    d = segtab_ref[1:2, :] - s0
    e = t + p + s0 + sf * d
    m = m_ref[...]
    mean_b = lax.dot(e, m, precision=lax.Precision.DEFAULT)
    e2_b = lax.dot(e * e, m, precision=lax.Precision.DEFAULT)
    var = e2_b - mean_b * mean_b
    out_ref[...] = (
        (e - mean_b) * lax.rsqrt(var + EPS) * gamma_ref[...] + beta_ref[...]
    )


def _ln_body_alias(tok_ref, pos_ref, seg_ref, segtab_ref, m_ref, gamma_ref,
                   beta_ref, prev_ref, out_ref):
    del prev_ref
    _ln_body(tok_ref, pos_ref, seg_ref, segtab_ref, m_ref, gamma_ref, beta_ref,
             out_ref)


def _ln_in_specs(k):
    return [
        pl.BlockSpec((R, DIM), lambda sb, b: (b * SB + sb, 0)),
        pl.BlockSpec((R, DIM), lambda sb, b: (sb, 0)),
        pl.BlockSpec((1, 1, R), lambda sb, b, k=k: (k * NSB + b * SB + sb, 0, 0)),
        pl.BlockSpec((2, DIM), lambda sb, b: (0, 0)),
        pl.BlockSpec((DIM, DIM), lambda sb, b: (0, 0)),
        pl.BlockSpec((1, DIM), lambda sb, b: (0, 0)),
        pl.BlockSpec((1, DIM), lambda sb, b: (0, 0)),
    ]


def _ln_slice(k, tok_e_s, pos_table, seg_f, seg_table, mmat, gamma2, beta2, prev):
    out_spec = pl.BlockSpec(
        (R, DIM), lambda sb, b, k=k: (k * NSB + b * SB + sb, 0)
    )
    out_shape = jax.ShapeDtypeStruct((N, DIM), jnp.float32)
    args = (tok_e_s, pos_table, seg_f, seg_table, mmat, gamma2, beta2)
    if prev is None:
        return pl.pallas_call(
            _ln_body,
            grid=(SB, B_SLICE),
            in_specs=_ln_in_specs(k),
            out_specs=out_spec,
            out_shape=out_shape,
        )(*args)
    return pl.pallas_call(
        _ln_body_alias,
        grid=(SB, B_SLICE),
        in_specs=_ln_in_specs(k) + [pl.BlockSpec(memory_space=pl.ANY)],
        out_specs=out_spec,
        out_shape=out_shape,
        input_output_aliases={7: 0},
    )(*args, prev)


def kernel(x, seg, tok_table, pos_table, seg_table, gamma, beta):
    idx = x.astype(jnp.int32).reshape(P, NW, CPW, CHUNK)
    seg_f = seg.astype(jnp.float32).reshape(P * NSB, 1, R)
    gamma2 = gamma.reshape(1, DIM)
    beta2 = beta.reshape(1, DIM)
    mmat = jnp.full((DIM, DIM), 1.0 / DIM, dtype=jnp.float32)

    tok_slices = [_tok_gathers[k](idx, tok_table) for k in range(P)]
    out = None
    for k in range(P):
        out = _ln_slice(
            k, tok_slices[k], pos_table, seg_f, seg_table, mmat, gamma2,
            beta2, out
        )
    return out.reshape(BATCH, SEQ, DIM)
